# baseline (speedup 1.0000x reference)
import functools

import jax
import jax.numpy as jnp
from jax import lax
from jax.experimental import pallas as pl
from jax.experimental.pallas import tpu as pltpu
from jax.experimental.pallas import tpu_sc as plsc

B = 16384
D = 32
L = 16
NC = 2
NS = 16
NW = NC * NS
BPW = B // NW

_mesh = plsc.VectorSubcoreMesh(core_axis_name="c", subcore_axis_name="s")

_GATHER_DNUMS = lax.GatherDimensionNumbers(
    offset_dims=(), collapsed_slice_dims=(0,), start_index_map=(0,))


def _perm(x, idx):
    return lax.gather(
        x, idx[:, None], dimension_numbers=_GATHER_DNUMS, slice_sizes=(1,),
        mode=lax.GatherScatterMode.PROMISE_IN_BOUNDS)


@functools.partial(
    pl.kernel,
    mesh=_mesh,
    out_type=jax.ShapeDtypeStruct((B,), jnp.float32),
    compiler_params=pltpu.CompilerParams(use_tc_tiling_on_sc=False),
    scratch_types=[
        pltpu.VMEM((BPW,), jnp.int32),
        pltpu.VMEM((BPW,), jnp.int32),
        pltpu.VMEM((BPW, D), jnp.float32),
        pltpu.VMEM((BPW, D), jnp.float32),
        pltpu.VMEM((BPW,), jnp.float32),
        pltpu.SemaphoreType.DMA,
    ],
)
def _dot_lookup(uid_hbm, iid_hbm, umem_hbm, imem_hbm, out_hbm,
                idx_u, idx_i, rows_u, rows_i, out_v, sem):
    wid = lax.axis_index("s") * NC + lax.axis_index("c")
    base = wid * BPW

    pltpu.sync_copy(uid_hbm.at[pl.ds(base, BPW)], idx_u)
    pltpu.sync_copy(iid_hbm.at[pl.ds(base, BPW)], idx_i)
    cu = pltpu.async_copy(umem_hbm.at[idx_u], rows_u, sem)
    ci = pltpu.async_copy(imem_hbm.at[idx_i], rows_i, sem)
    cu.wait()
    ci.wait()

    lane = lax.iota(jnp.int32, L)
    xor_perms = {s: jnp.bitwise_xor(lane, s) for s in (8, 4, 2, 1)}
    bitrev = ((lane & 1) << 3) | ((lane & 2) << 1) | ((lane & 4) >> 1) | ((lane & 8) >> 3)

    def merge(x, y, s):
        z = jnp.where((lane & s) == 0,
                      x + _perm(x, xor_perms[s]),
                      y + _perm(y, xor_perms[s]))
        return z

    def block(bi, carry):
        vecs = []
        for j in range(L):
            r = bi * L + j
            u0 = rows_u[r, pl.ds(0, L)]
            u1 = rows_u[r, pl.ds(L, L)]
            v0 = rows_i[r, pl.ds(0, L)]
            v1 = rows_i[r, pl.ds(L, L)]
            vecs.append(u0 * v0 + u1 * v1)
        for s in (8, 4, 2, 1):
            vecs = [merge(vecs[2 * j], vecs[2 * j + 1], s)
                    for j in range(len(vecs) // 2)]
        out_v[pl.ds(bi * L, L)] = _perm(vecs[0], bitrev)
        return carry

    lax.fori_loop(0, BPW // L, block, 0)

    pltpu.sync_copy(out_v, out_hbm.at[pl.ds(base, BPW)])


def kernel(userids, itemids, user_memory, item_memory):
    uid = userids.astype(jnp.int32)
    iid = itemids.astype(jnp.int32)
    return _dot_lookup(uid, iid, user_memory, item_memory)
